# baseline (speedup 1.0000x reference)
import functools

import jax
import jax.numpy as jnp
from jax import lax
from jax.experimental import pallas as pl
from jax.experimental.pallas import tpu as pltpu
from jax.experimental.pallas import tpu_sc as plsc

_CHUNK = 512


@functools.lru_cache(maxsize=None)
def _make_gather(num_rows: int, dim: int, batch: int):
    info = plsc.get_sparse_core_info()
    nw = info.num_cores * info.num_subcores
    assert batch % (8 * nw) == 0
    b_per_w = batch // nw
    assert b_per_w % _CHUNK == 0
    n_chunks = b_per_w // _CHUNK
    assert n_chunks % 2 == 0
    mesh = plsc.VectorSubcoreMesh(core_axis_name="c", subcore_axis_name="s")

    @functools.partial(
        pl.kernel,
        mesh=mesh,
        compiler_params=pltpu.CompilerParams(use_tc_tiling_on_sc=False),
        out_type=jax.ShapeDtypeStruct((batch, dim), jnp.float32),
        scratch_types=[
            pltpu.VMEM((b_per_w,), jnp.int32),
            pltpu.VMEM((_CHUNK, dim), jnp.float32),
            pltpu.VMEM((_CHUNK, dim), jnp.float32),
            pltpu.SemaphoreType.DMA,
            pltpu.SemaphoreType.DMA,
        ],
    )
    def gather_kernel(table_hbm, idx_hbm, out_hbm, idx_v, rows0, rows1,
                      sem0, sem1):
        wid = lax.axis_index("s") * info.num_cores + lax.axis_index("c")
        base = pl.multiple_of(wid * b_per_w, 8)
        pltpu.sync_copy(idx_hbm.at[pl.ds(base, b_per_w)], idx_v)

        bufs = ((rows0, sem0), (rows1, sem1))

        def start(g, rows, sem):
            off = pl.multiple_of(g * _CHUNK, 8)
            pltpu.async_copy(table_hbm.at[idx_v.at[pl.ds(off, _CHUNK)]],
                             rows, sem)

        def drain_and_emit(g, rows, sem):
            pltpu.make_async_copy(table_hbm.at[idx_v.at[pl.ds(0, _CHUNK)]],
                                  rows, sem).wait()
            off = pl.multiple_of(base + g * _CHUNK, 8)
            pltpu.sync_copy(rows, out_hbm.at[pl.ds(off, _CHUNK)])

        start(0, rows0, sem0)
        start(1, rows1, sem1)

        def pair_body(p, carry):
            for b, (rows, sem) in enumerate(bufs):
                g = 2 * p + b
                drain_and_emit(g, rows, sem)

                @pl.when(g + 2 < n_chunks)
                def _():
                    start(g + 2, rows, sem)
            return carry

        lax.fori_loop(0, n_chunks // 2, pair_body, 0)

    return gather_kernel


def kernel(ids, weight):
    batch, seq = ids.shape
    num_rows, dim = weight.shape
    flat_ids = ids.reshape(-1).astype(jnp.int32)
    gather = _make_gather(num_rows, dim, flat_ids.shape[0])
    out = gather(weight, flat_ids)
    return out.reshape(batch, seq, dim)
